# baseline (speedup 1.0000x reference)
import dataclasses
import functools

import jax
import jax.numpy as jnp
from jax import lax
from jax.experimental import pallas as pl
from jax.experimental.pallas import tpu as pltpu
from jax.experimental.pallas import tpu_sc as plsc

N = 10000
D = 128
H = 8
DH = 16
E = 320000
ROWS = 1000
GRID = N // ROWS
CHUNK = 32
NCHUNK = E // CHUNK
NTILES = 32
SUPER = 8
NBLOCKS = NCHUNK // SUPER
NSUPER = (NBLOCKS + NTILES - 1) // NTILES
NPAD = 10240
STRIPE = NPAD // 16
NALL = NPAD + NPAD // 8


def _ln(x, g, b):
    mu = jnp.mean(x, axis=1, keepdims=True)
    xc = x - mu
    var = jnp.mean(xc * xc, axis=1, keepdims=True)
    return xc * lax.rsqrt(var + 1e-5) * g + b


def _lane_bcast(v, idx):
    return lax.gather(
        v, idx[:, None],
        lax.GatherDimensionNumbers(
            offset_dims=(), collapsed_slice_dims=(0,), start_index_map=(0,)),
        (1,), mode=lax.GatherScatterMode.PROMISE_IN_BOUNDS)


def _head_sum_matrix():
    k_iota = lax.broadcasted_iota(jnp.int32, (D, H), 0)
    h_iota = lax.broadcasted_iota(jnp.int32, (D, H), 1)
    return jnp.where(k_iota // DH == h_iota, 1.0, 0.0).astype(jnp.float32)


def _pre_body(h_ref, g1_ref, b1_ref, wfc_ref, ft_ref, hn_ref, gqb_ref, gmax):
    i = pl.program_id(0)
    x = h_ref[...]
    hn = _ln(x, g1_ref[...], b1_ref[...])
    hn_ref[...] = hn
    ft = jnp.dot(hn, wfc_ref[...], preferred_element_type=jnp.float32) * 0.5
    ft_ref[...] = ft
    s_mat = _head_sum_matrix()
    g2 = jnp.dot(ft * ft, s_mat, preferred_element_type=jnp.float32)
    bm = jnp.max(g2, axis=0, keepdims=True)

    @pl.when(i == 0)
    def _():
        gmax[...] = bm

    @pl.when(i > 0)
    def _():
        gmax[...] = jnp.maximum(gmax[...], bm)

    @pl.when(i == GRID - 1)
    def _():
        gqb_ref[...] = jnp.concatenate(
            [gmax[...], jnp.zeros((1, 8), jnp.float32)], axis=1)


def _sc_edge(ft, src, dst, gqb):
    mesh = plsc.VectorSubcoreMesh(core_axis_name="c", subcore_axis_name="s")
    cp = pltpu.CompilerParams()
    if "needs_layout_passes" in pltpu.CompilerParams.__dataclass_fields__:
        cp = dataclasses.replace(cp, needs_layout_passes=False)

    @functools.partial(
        pl.kernel,
        compiler_params=cp,
        out_type=[
            jax.ShapeDtypeStruct((2, N, D), jnp.float32),
            jax.ShapeDtypeStruct((2, NPAD // 8, D), jnp.float32),
        ],
        mesh=mesh,
        scratch_types=[
            pltpu.VMEM((SUPER * CHUNK,), jnp.int32),
            pltpu.VMEM((SUPER * CHUNK,), jnp.int32),
            pltpu.VMEM((2 * CHUNK,), jnp.int32),
            pltpu.VMEM((2 * CHUNK,), jnp.int32),
            pltpu.VMEM((2 * CHUNK,), jnp.int32),
            pltpu.VMEM((2 * CHUNK,), jnp.int32),
            pltpu.VMEM((2 * CHUNK, D), jnp.float32),
            pltpu.VMEM((2 * CHUNK, D), jnp.float32),
            pltpu.VMEM((2 * CHUNK, D), jnp.float32),
            pltpu.VMEM((2 * CHUNK, D), jnp.float32),
            pltpu.VMEM((16,), jnp.float32),
            pltpu.VMEM_SHARED((NALL, D), jnp.float32),
            pltpu.SemaphoreType.DMA,
            pltpu.SemaphoreType.DMA,
            pltpu.SemaphoreType.DMA,
            pltpu.SemaphoreType.DMA,
        ],
    )
    def k(ft_hbm, src_hbm, dst_hbm, gq_hbm, outm_hbm, oute_hbm,
          sidxbuf, didxbuf, usdidx0, usdidx1, didxall0, didxall1,
          usd0, usd1, ostg0, ostg1, gqv, acc,
          semg0, semg1, sems0, sems1):
        usdidx = [usdidx0, usdidx1]
        didxall = [didxall0, didxall1]
        usd = [usd0, usd1]
        ostg = [ostg0, ostg1]
        semg = [semg0, semg1]
        sems = [sems0, sems1]
        cid = lax.axis_index("c")
        sid = lax.axis_index("s")
        wid = sid * 2 + cid
        pltpu.sync_copy(gq_hbm, gqv)
        gqvec = gqv[...]
        lane = lax.iota(jnp.int32, 16)
        zero16 = jnp.zeros((16,), jnp.float32)

        @pl.loop(0, 2 * CHUNK)
        def _(r):
            for cb in range(D // 16):
                ostg0[r, pl.ds(cb * 16, 16)] = zero16
                ostg1[r, pl.ds(cb * 16, 16)] = zero16

        abase = sid * (NALL // 16)
        for j in range(11):
            pltpu.sync_copy(ostg0, acc.at[pl.ds(abase + j * 64, 64)])
        pltpu.sync_copy(ostg0.at[pl.ds(0, 16)], acc.at[pl.ds(abase + 704, 16)])
        plsc.subcore_barrier()

        def build_usdidx(q, s):
            qo = q * CHUNK
            for t in range(CHUNK // 16):
                usdidx[s][pl.ds(t * 16, 16)] = sidxbuf[pl.ds(qo + t * 16, 16)]
                usdidx[s][pl.ds(CHUNK + t * 16, 16)] = (
                    didxbuf[pl.ds(qo + t * 16, 16)])

        def build_didxall(q, s):
            qo = q * CHUNK
            for t in range(CHUNK // 16):
                dv = didxbuf[pl.ds(qo + t * 16, 16)]
                didxall[s][pl.ds(t * 16, 16)] = dv
                didxall[s][pl.ds(CHUNK + t * 16, 16)] = (
                    lax.shift_right_logical(dv, 3) + NPAD)

        def compute(q, s):
            qo = q * CHUNK

            @plsc.parallel_loop(0, CHUNK, 1, unroll=3)
            def _(i):
                evec = zero16
                avecs = []
                for hh in range(H):
                    a = usd[s][i, pl.ds(hh * DH, DH)]
                    b = usd[s][CHUNK + i, pl.ds(hh * DH, DH)]
                    avecs.append(a)
                    sv = jnp.sum(a * b)
                    evec = jnp.where(lane == hh, sv, evec)
                ee = jnp.exp(evec - gqvec)
                bb = pl.multiple_of((i // 16) * 16, 16)
                gv = didxall[s][pl.ds(bb, 16)] & 7
                grp = jnp.sum(jnp.where(lane == (i & 15), gv, 0))
                for g in range(8):
                    ostg[s][CHUNK + i, pl.ds(g * DH, DH)] = zero16
                ostg[s][CHUNK + i, pl.ds(grp * DH, DH)] = ee
                for hh in range(H):
                    bc = _lane_bcast(ee, lane * 0 + hh)
                    ostg[s][i, pl.ds(hh * DH, DH)] = avecs[hh] * bc

        @pl.loop(0, NSUPER)
        def _(sup):
            blk = wid + sup * NTILES

            @pl.when(blk < NBLOCKS)
            def _():
                sbase = blk * SUPER * CHUNK
                pltpu.sync_copy(src_hbm.at[pl.ds(sbase, SUPER * CHUNK)],
                                sidxbuf)
                pltpu.sync_copy(dst_hbm.at[pl.ds(sbase, SUPER * CHUNK)],
                                didxbuf)
                gh = [None, None]
                sh = [None, None]
                build_usdidx(0, 0)
                gh[0] = pltpu.async_copy(ft_hbm.at[usdidx[0]], usd[0], semg[0])
                for q in range(SUPER):
                    s = q & 1
                    ns = 1 - s
                    gh[s].wait()
                    if q < SUPER - 1:
                        build_usdidx(q + 1, ns)
                        gh[ns] = pltpu.async_copy(
                            ft_hbm.at[usdidx[ns]], usd[ns], semg[ns])
                    if sh[s] is not None:
                        sh[s].wait()
                        sh[s] = None
                    build_didxall(q, s)
                    compute(q, s)
                    sh[s] = pltpu.async_copy(
                        ostg[s], acc.at[didxall[s]], sems[s], add=True)
                sh[0].wait()
                sh[1].wait()

        plsc.subcore_barrier()
        base = sid * STRIPE
        last = N - 15 * STRIPE

        @pl.when(sid < 15)
        def _():
            pltpu.sync_copy(acc.at[pl.ds(base, STRIPE)],
                            outm_hbm.at[cid, pl.ds(base, STRIPE)])

        @pl.when(sid == 15)
        def _():
            pltpu.sync_copy(acc.at[pl.ds(15 * STRIPE, last)],
                            outm_hbm.at[cid, pl.ds(15 * STRIPE, last)])

        erows = NPAD // 8 // 16
        pltpu.sync_copy(acc.at[pl.ds(NPAD + sid * erows, erows)],
                        oute_hbm.at[cid, pl.ds(sid * erows, erows)])

    return k(ft, src, dst, gqb)


def _post_body(pm_ref, pe_ref, hn_ref, wr_ref, br_ref, g2_ref, b2_ref,
               w1_ref, bb1_ref, w2_ref, bb2_ref, out_ref):
    aggnum = pm_ref[0] + pm_ref[1]
    esum = (pe_ref[0] + pe_ref[1])[:, :H]
    inv = jnp.where(esum > 0.0, 2.0 / esum, 0.0)
    invrep = jnp.dot(inv, _head_sum_matrix().T,
                     preferred_element_type=jnp.float32)
    agg = aggnum * invrep
    h2 = (jnp.dot(agg, wr_ref[...], preferred_element_type=jnp.float32)
          + br_ref[...] + hn_ref[...])
    h2n = _ln(h2, g2_ref[...], b2_ref[...])
    u = jnp.dot(h2n, w1_ref[...], preferred_element_type=jnp.float32) + bb1_ref[...]
    u = jnp.where(u > 0.0, u, jnp.exp(u) - 1.0)
    v = jnp.dot(u, w2_ref[...], preferred_element_type=jnp.float32) + bb2_ref[...]
    v = jnp.where(v > 0.0, v, jnp.exp(v) - 1.0)
    out_ref[...] = v + h2n


def kernel(h, edge_index, ln1_g, ln1_b, W_fc, Wr, br, ln2_g, ln2_b, W1, b1, W2, b2):
    ft, hn, gqb = pl.pallas_call(
        _pre_body,
        grid=(GRID,),
        in_specs=[
            pl.BlockSpec((ROWS, D), lambda i: (i, 0)),
            pl.BlockSpec((1, D), lambda i: (0, 0)),
            pl.BlockSpec((1, D), lambda i: (0, 0)),
            pl.BlockSpec((D, D), lambda i: (0, 0)),
        ],
        out_specs=[
            pl.BlockSpec((ROWS, D), lambda i: (i, 0)),
            pl.BlockSpec((ROWS, D), lambda i: (i, 0)),
            pl.BlockSpec((1, 16), lambda i: (0, 0)),
        ],
        out_shape=[
            jax.ShapeDtypeStruct((N, D), jnp.float32),
            jax.ShapeDtypeStruct((N, D), jnp.float32),
            jax.ShapeDtypeStruct((1, 16), jnp.float32),
        ],
        scratch_shapes=[pltpu.VMEM((1, H), jnp.float32)],
    )(h, ln1_g.reshape(1, D), ln1_b.reshape(1, D), W_fc)

    pm, pe_packed = _sc_edge(ft, edge_index[0], edge_index[1], gqb.reshape(16))
    pe = pe_packed.reshape(2, NPAD, 16)

    y = pl.pallas_call(
        _post_body,
        grid=(GRID,),
        in_specs=[
            pl.BlockSpec((2, ROWS, D), lambda i: (0, i, 0)),
            pl.BlockSpec((2, ROWS, 16), lambda i: (0, i, 0)),
            pl.BlockSpec((ROWS, D), lambda i: (i, 0)),
            pl.BlockSpec((D, D), lambda i: (0, 0)),
            pl.BlockSpec((1, D), lambda i: (0, 0)),
            pl.BlockSpec((1, D), lambda i: (0, 0)),
            pl.BlockSpec((1, D), lambda i: (0, 0)),
            pl.BlockSpec((D, 4 * D), lambda i: (0, 0)),
            pl.BlockSpec((1, 4 * D), lambda i: (0, 0)),
            pl.BlockSpec((4 * D, D), lambda i: (0, 0)),
            pl.BlockSpec((1, D), lambda i: (0, 0)),
        ],
        out_specs=pl.BlockSpec((ROWS, D), lambda i: (i, 0)),
        out_shape=jax.ShapeDtypeStruct((N, D), jnp.float32),
    )(pm, pe, hn, Wr, br.reshape(1, D), ln2_g.reshape(1, D),
      ln2_b.reshape(1, D), W1, b1.reshape(1, 4 * D), W2, b2.reshape(1, D))
    return y


# speedup vs baseline: 1.0303x; 1.0303x over previous
import dataclasses
import functools

import jax
import jax.numpy as jnp
from jax import lax
from jax.experimental import pallas as pl
from jax.experimental.pallas import tpu as pltpu
from jax.experimental.pallas import tpu_sc as plsc

N = 10000
D = 128
H = 8
DH = 16
E = 320000
ROWS = 1000
GRID = N // ROWS
CHUNK = 32
NCHUNK = E // CHUNK
NTILES = 32
SUPER = 8
NBLOCKS = NCHUNK // SUPER
NSUPER = (NBLOCKS + NTILES - 1) // NTILES
NPAD = 10240
STRIPE = NPAD // 16
NALL = NPAD + NPAD // 8


def _ln(x, g, b):
    mu = jnp.mean(x, axis=1, keepdims=True)
    xc = x - mu
    var = jnp.mean(xc * xc, axis=1, keepdims=True)
    return xc * lax.rsqrt(var + 1e-5) * g + b


def _lane_bcast(v, idx):
    return lax.gather(
        v, idx[:, None],
        lax.GatherDimensionNumbers(
            offset_dims=(), collapsed_slice_dims=(0,), start_index_map=(0,)),
        (1,), mode=lax.GatherScatterMode.PROMISE_IN_BOUNDS)


def _head_sum_matrix():
    k_iota = lax.broadcasted_iota(jnp.int32, (D, H), 0)
    h_iota = lax.broadcasted_iota(jnp.int32, (D, H), 1)
    return jnp.where(k_iota // DH == h_iota, 1.0, 0.0).astype(jnp.float32)


def _pre_body(h_ref, g1_ref, b1_ref, wfc_ref, ft_ref, hn_ref, gqb_ref, gmax):
    i = pl.program_id(0)
    x = h_ref[...]
    hn = _ln(x, g1_ref[...], b1_ref[...])
    hn_ref[...] = hn
    ft = jnp.dot(hn, wfc_ref[...], preferred_element_type=jnp.float32) * 0.5
    ft_ref[...] = ft
    s_mat = _head_sum_matrix()
    g2 = jnp.dot(ft * ft, s_mat, preferred_element_type=jnp.float32)
    bm = jnp.max(g2, axis=0, keepdims=True)

    @pl.when(i == 0)
    def _():
        gmax[...] = bm

    @pl.when(i > 0)
    def _():
        gmax[...] = jnp.maximum(gmax[...], bm)

    @pl.when(i == GRID - 1)
    def _():
        gqb_ref[...] = jnp.concatenate(
            [gmax[...], jnp.zeros((1, 8), jnp.float32)], axis=1)


def _sc_edge(ft, src, dst, gqb):
    mesh = plsc.VectorSubcoreMesh(core_axis_name="c", subcore_axis_name="s")
    cp = pltpu.CompilerParams()
    if "needs_layout_passes" in pltpu.CompilerParams.__dataclass_fields__:
        cp = dataclasses.replace(cp, needs_layout_passes=False)

    @functools.partial(
        pl.kernel,
        compiler_params=cp,
        out_type=[
            jax.ShapeDtypeStruct((2, N, D), jnp.float32),
            jax.ShapeDtypeStruct((2, NPAD // 8, D), jnp.float32),
        ],
        mesh=mesh,
        scratch_types=[
            pltpu.VMEM((SUPER * CHUNK,), jnp.int32),
            pltpu.VMEM((SUPER * CHUNK,), jnp.int32),
            pltpu.VMEM((2 * CHUNK,), jnp.int32),
            pltpu.VMEM((2 * CHUNK,), jnp.int32),
            pltpu.VMEM((2 * CHUNK,), jnp.int32),
            pltpu.VMEM((2 * CHUNK,), jnp.int32),
            pltpu.VMEM((2 * CHUNK, D), jnp.float32),
            pltpu.VMEM((2 * CHUNK, D), jnp.float32),
            pltpu.VMEM((2 * CHUNK, D), jnp.float32),
            pltpu.VMEM((2 * CHUNK, D), jnp.float32),
            pltpu.VMEM((16,), jnp.float32),
            pltpu.VMEM_SHARED((NALL, D), jnp.float32),
            pltpu.SemaphoreType.DMA,
            pltpu.SemaphoreType.DMA,
            pltpu.SemaphoreType.DMA,
            pltpu.SemaphoreType.DMA,
        ],
    )
    def k(ft_hbm, src_hbm, dst_hbm, gq_hbm, outm_hbm, oute_hbm,
          sidxbuf, didxbuf, usdidx0, usdidx1, didxall0, didxall1,
          usd0, usd1, ostg0, ostg1, gqv, acc,
          semg0, semg1, sems0, sems1):
        usdidx = [usdidx0, usdidx1]
        didxall = [didxall0, didxall1]
        usd = [usd0, usd1]
        ostg = [ostg0, ostg1]
        semg = [semg0, semg1]
        sems = [sems0, sems1]
        cid = lax.axis_index("c")
        sid = lax.axis_index("s")
        wid = sid * 2 + cid
        pltpu.sync_copy(gq_hbm, gqv)
        gqvec = gqv[...]
        lane = lax.iota(jnp.int32, 16)
        zero16 = jnp.zeros((16,), jnp.float32)

        @pl.loop(0, 2 * CHUNK)
        def _(r):
            for cb in range(D // 16):
                ostg0[r, pl.ds(cb * 16, 16)] = zero16
                ostg1[r, pl.ds(cb * 16, 16)] = zero16

        abase = sid * (NALL // 16)
        for j in range(11):
            pltpu.sync_copy(ostg0, acc.at[pl.ds(abase + j * 64, 64)])
        pltpu.sync_copy(ostg0.at[pl.ds(0, 16)], acc.at[pl.ds(abase + 704, 16)])
        plsc.subcore_barrier()

        def build_usdidx(q, s):
            qo = q * CHUNK
            for t in range(CHUNK // 16):
                usdidx[s][pl.ds(t * 16, 16)] = sidxbuf[pl.ds(qo + t * 16, 16)]
                usdidx[s][pl.ds(CHUNK + t * 16, 16)] = (
                    didxbuf[pl.ds(qo + t * 16, 16)])

        def build_didxall(q, s):
            qo = q * CHUNK
            for t in range(CHUNK // 16):
                dv = didxbuf[pl.ds(qo + t * 16, 16)]
                didxall[s][pl.ds(t * 16, 16)] = dv
                didxall[s][pl.ds(CHUNK + t * 16, 16)] = (
                    lax.shift_right_logical(dv, 3) + NPAD)

        def compute(q, s):
            qo = q * CHUNK

            @plsc.parallel_loop(0, CHUNK, 1, unroll=2)
            def _(i):
                evec = zero16
                avecs = []
                for hh in range(H):
                    a = usd[s][i, pl.ds(hh * DH, DH)]
                    b = usd[s][CHUNK + i, pl.ds(hh * DH, DH)]
                    avecs.append(a)
                    sv = jnp.sum(a * b)
                    evec = jnp.where(lane == hh, sv, evec)
                ee = jnp.exp(evec - gqvec)
                bb = pl.multiple_of((i // 16) * 16, 16)
                gv = didxall[s][pl.ds(bb, 16)] & 7
                grp = jnp.sum(jnp.where(lane == (i & 15), gv, 0))
                for g in range(8):
                    ostg[s][CHUNK + i, pl.ds(g * DH, DH)] = zero16
                ostg[s][CHUNK + i, pl.ds(grp * DH, DH)] = ee
                for hh in range(H):
                    bc = _lane_bcast(ee, lane * 0 + hh)
                    ostg[s][i, pl.ds(hh * DH, DH)] = avecs[hh] * bc

        @pl.loop(0, NSUPER)
        def _(sup):
            blk = wid + sup * NTILES

            @pl.when(blk < NBLOCKS)
            def _():
                sbase = blk * SUPER * CHUNK
                pltpu.sync_copy(src_hbm.at[pl.ds(sbase, SUPER * CHUNK)],
                                sidxbuf)
                pltpu.sync_copy(dst_hbm.at[pl.ds(sbase, SUPER * CHUNK)],
                                didxbuf)
                gh = [None, None]
                sh = [None, None]
                build_usdidx(0, 0)
                gh[0] = pltpu.async_copy(ft_hbm.at[usdidx[0]], usd[0], semg[0])
                for q in range(SUPER):
                    s = q & 1
                    ns = 1 - s
                    gh[s].wait()
                    if q < SUPER - 1:
                        build_usdidx(q + 1, ns)
                        gh[ns] = pltpu.async_copy(
                            ft_hbm.at[usdidx[ns]], usd[ns], semg[ns])
                    if sh[s] is not None:
                        sh[s].wait()
                        sh[s] = None
                    build_didxall(q, s)
                    compute(q, s)
                    sh[s] = pltpu.async_copy(
                        ostg[s], acc.at[didxall[s]], sems[s], add=True)
                sh[0].wait()
                sh[1].wait()

        plsc.subcore_barrier()
        base = sid * STRIPE
        last = N - 15 * STRIPE

        @pl.when(sid < 15)
        def _():
            pltpu.sync_copy(acc.at[pl.ds(base, STRIPE)],
                            outm_hbm.at[cid, pl.ds(base, STRIPE)])

        @pl.when(sid == 15)
        def _():
            pltpu.sync_copy(acc.at[pl.ds(15 * STRIPE, last)],
                            outm_hbm.at[cid, pl.ds(15 * STRIPE, last)])

        erows = NPAD // 8 // 16
        pltpu.sync_copy(acc.at[pl.ds(NPAD + sid * erows, erows)],
                        oute_hbm.at[cid, pl.ds(sid * erows, erows)])

    return k(ft, src, dst, gqb)


def _post_body(pm_ref, pe_ref, hn_ref, wr_ref, br_ref, g2_ref, b2_ref,
               w1_ref, bb1_ref, w2_ref, bb2_ref, out_ref):
    aggnum = pm_ref[0] + pm_ref[1]
    esum = (pe_ref[0] + pe_ref[1])[:, :H]
    inv = jnp.where(esum > 0.0, 2.0 / esum, 0.0)
    invrep = jnp.dot(inv, _head_sum_matrix().T,
                     preferred_element_type=jnp.float32)
    agg = aggnum * invrep
    h2 = (jnp.dot(agg, wr_ref[...], preferred_element_type=jnp.float32)
          + br_ref[...] + hn_ref[...])
    h2n = _ln(h2, g2_ref[...], b2_ref[...])
    u = jnp.dot(h2n, w1_ref[...], preferred_element_type=jnp.float32) + bb1_ref[...]
    u = jnp.where(u > 0.0, u, jnp.exp(u) - 1.0)
    v = jnp.dot(u, w2_ref[...], preferred_element_type=jnp.float32) + bb2_ref[...]
    v = jnp.where(v > 0.0, v, jnp.exp(v) - 1.0)
    out_ref[...] = v + h2n


def kernel(h, edge_index, ln1_g, ln1_b, W_fc, Wr, br, ln2_g, ln2_b, W1, b1, W2, b2):
    ft, hn, gqb = pl.pallas_call(
        _pre_body,
        grid=(GRID,),
        in_specs=[
            pl.BlockSpec((ROWS, D), lambda i: (i, 0)),
            pl.BlockSpec((1, D), lambda i: (0, 0)),
            pl.BlockSpec((1, D), lambda i: (0, 0)),
            pl.BlockSpec((D, D), lambda i: (0, 0)),
        ],
        out_specs=[
            pl.BlockSpec((ROWS, D), lambda i: (i, 0)),
            pl.BlockSpec((ROWS, D), lambda i: (i, 0)),
            pl.BlockSpec((1, 16), lambda i: (0, 0)),
        ],
        out_shape=[
            jax.ShapeDtypeStruct((N, D), jnp.float32),
            jax.ShapeDtypeStruct((N, D), jnp.float32),
            jax.ShapeDtypeStruct((1, 16), jnp.float32),
        ],
        scratch_shapes=[pltpu.VMEM((1, H), jnp.float32)],
    )(h, ln1_g.reshape(1, D), ln1_b.reshape(1, D), W_fc)

    pm, pe_packed = _sc_edge(ft, edge_index[0], edge_index[1], gqb.reshape(16))
    pe = pe_packed.reshape(2, NPAD, 16)

    y = pl.pallas_call(
        _post_body,
        grid=(GRID,),
        in_specs=[
            pl.BlockSpec((2, ROWS, D), lambda i: (0, i, 0)),
            pl.BlockSpec((2, ROWS, 16), lambda i: (0, i, 0)),
            pl.BlockSpec((ROWS, D), lambda i: (i, 0)),
            pl.BlockSpec((D, D), lambda i: (0, 0)),
            pl.BlockSpec((1, D), lambda i: (0, 0)),
            pl.BlockSpec((1, D), lambda i: (0, 0)),
            pl.BlockSpec((1, D), lambda i: (0, 0)),
            pl.BlockSpec((D, 4 * D), lambda i: (0, 0)),
            pl.BlockSpec((1, 4 * D), lambda i: (0, 0)),
            pl.BlockSpec((4 * D, D), lambda i: (0, 0)),
            pl.BlockSpec((1, D), lambda i: (0, 0)),
        ],
        out_specs=pl.BlockSpec((ROWS, D), lambda i: (i, 0)),
        out_shape=jax.ShapeDtypeStruct((N, D), jnp.float32),
    )(pm, pe, hn, Wr, br.reshape(1, D), ln2_g.reshape(1, D),
      ln2_b.reshape(1, D), W1, b1.reshape(1, 4 * D), W2, b2.reshape(1, D))
    return y
